# baseline (speedup 1.0000x reference)
import functools

import jax
import jax.numpy as jnp
from jax import lax
from jax.experimental import pallas as pl
from jax.experimental.pallas import tpu as pltpu
from jax.experimental.pallas import tpu_sc as plsc

NC, NS = 2, 16
NW = NC * NS
LANE = 16
VPAD = 128

CB = 25
RB = 128
GRP = 1024
RPW = RB // NW
RCH = 2
ITERS = CB * (RPW // RCH)
NBUF = 2


def _fold_linear(emb_pad, W, b):
    O = W.shape[0]

    def body(emb_ref, w_ref, b_ref, t_ref):
        t_ref[...] = lax.dot_general(
            w_ref[...], emb_ref[...], (((1,), (1,)), ((), ())),
            preferred_element_type=jnp.float32) + b_ref[...]

    return pl.pallas_call(
        body,
        out_shape=jax.ShapeDtypeStruct((O, VPAD), jnp.float32),
    )(emb_pad, W, b.reshape(O, 1))


def _sc_lookup(table4, ids_phys):
    O = table4.shape[0]
    mesh = plsc.VectorSubcoreMesh(core_axis_name="c", subcore_axis_name="s")

    @functools.partial(
        pl.kernel,
        out_type=jax.ShapeDtypeStruct((8 * CB, RB, O * 128), jnp.float32),
        mesh=mesh,
        scratch_types=[
            pltpu.VMEM((O, VPAD), jnp.float32),
            pltpu.VMEM((O * VPAD * LANE,), jnp.float32),
            [pltpu.VMEM((RCH, GRP), jnp.int32) for _ in range(NBUF)],
            [pltpu.VMEM((8, RCH, O * 128), jnp.float32) for _ in range(NBUF)],
            [pltpu.SemaphoreType.DMA for _ in range(NBUF)],
            [pltpu.SemaphoreType.DMA for _ in range(NBUF)],
        ],
        compiler_params=pltpu.CompilerParams(
            use_tc_tiling_on_sc=False, needs_layout_passes=False),
    )
    def body(t_ref, ids_ref, out_ref, t_v, ts_v, in_v, out_v, in_sem, out_sem):
        wid = lax.axis_index("s") * NC + lax.axis_index("c")
        pltpu.sync_copy(t_ref, t_v)

        def spread(s, carry):
            r0 = s * LANE
            for c in range(O):
                vals16 = t_v[c, pl.ds(r0, LANE)]
                for k in range(LANE):
                    ts_v[pl.ds((r0 + k) * (O * LANE) + c * LANE, LANE)] = (
                        jnp.broadcast_to(vals16[k], (LANE,)))
            return carry

        lax.fori_loop(0, VPAD // LANE, spread, 0)
        offs = [lax.iota(jnp.int32, LANE) + c * LANE for c in range(O)]

        def coords(it):
            cblk = it // (RPW // RCH)
            rblk0 = wid * RPW + (it % (RPW // RCH)) * RCH
            return cblk, rblk0

        def start_in(it, buf):
            cblk, rblk0 = coords(it)
            pltpu.async_copy(
                ids_ref.at[cblk, pl.ds(rblk0, RCH)], in_v[buf], in_sem[buf])

        def wait_in(buf):
            pltpu.make_async_copy(
                ids_ref.at[0, pl.ds(0, RCH)], in_v[buf], in_sem[buf]).wait()

        def start_out(it, buf):
            cblk, rblk0 = coords(it)
            pltpu.async_copy(
                out_v[buf], out_ref.at[pl.ds(8 * cblk, 8), pl.ds(rblk0, RCH)],
                out_sem[buf])

        def wait_out(buf):
            pltpu.make_async_copy(
                out_v[buf], out_ref.at[pl.ds(0, 8), pl.ds(0, RCH)],
                out_sem[buf]).wait()

        for buf in range(NBUF):
            start_in(buf, buf)

        def step(i, carry):
            for buf in range(NBUF):
                it = NBUF * i + buf
                wait_in(buf)

                @pl.when(it >= NBUF)
                def _():
                    wait_out(buf)

                for j in range(RCH):
                    @plsc.parallel_loop(0, GRP // LANE, unroll=8)
                    def _(s, _j=j, _buf=buf):
                        ids16 = in_v[_buf][_j, pl.ds(s * LANE, LANE)]
                        base = ids16 * (O * LANE)
                        row = s // 8
                        b0 = (s % 8) * LANE
                        for c in range(O):
                            vals = plsc.load_gather(ts_v, [base + offs[c]])
                            out_v[_buf][row, _j, pl.ds(c * 128 + b0, LANE)] = (
                                vals)
                start_out(it, buf)

                @pl.when(it + NBUF < ITERS)
                def _():
                    start_in(it + NBUF, buf)

            return carry

        lax.fori_loop(0, ITERS // NBUF, step, 0)
        for buf in range(NBUF):
            wait_out(buf)

    return body(table4, ids_phys)


def kernel(input_ids, emb_table, W, b):
    Bsz, Lseq = input_ids.shape
    V, D = emb_table.shape
    O = W.shape[0]
    emb_pad = jnp.zeros((VPAD, D), jnp.float32).at[:V].set(emb_table)
    table4 = _fold_linear(emb_pad, W, b)
    ids_phys = (input_ids.reshape(RB, 128, CB, 8)
                .transpose(2, 0, 3, 1).reshape(CB, RB, GRP))
    out_phys = _sc_lookup(table4, ids_phys)
    out = (out_phys.reshape(Lseq, RB, O, 128)
           .transpose(1, 3, 0, 2).reshape(Bsz, Lseq, O))
    return out


# speedup vs baseline: 1.0578x; 1.0578x over previous
import functools

import jax
import jax.numpy as jnp
from jax import lax
from jax.experimental import pallas as pl
from jax.experimental.pallas import tpu as pltpu
from jax.experimental.pallas import tpu_sc as plsc

NC, NS = 2, 16
NW = NC * NS
LANE = 16
VPAD = 128

CB = 25
RB = 128
GRP = 1024
CCH = 5
RPW = RB // NW
RCH = 2
ITERS = (RPW // RCH) * (CB // CCH)
NBUF = 2


def _fold_linear(emb_pad, W, b):
    O = W.shape[0]

    def body(emb_ref, w_ref, b_ref, t_ref):
        t_ref[...] = lax.dot_general(
            w_ref[...], emb_ref[...], (((1,), (1,)), ((), ())),
            preferred_element_type=jnp.float32) + b_ref[...]

    return pl.pallas_call(
        body,
        out_shape=jax.ShapeDtypeStruct((O, VPAD), jnp.float32),
    )(emb_pad, W, b.reshape(O, 1))


def _sc_lookup(table4, ids_phys):
    O = table4.shape[0]
    mesh = plsc.VectorSubcoreMesh(core_axis_name="c", subcore_axis_name="s")

    @functools.partial(
        pl.kernel,
        out_type=jax.ShapeDtypeStruct((8 * CB, RB, O * 128), jnp.float32),
        mesh=mesh,
        scratch_types=[
            pltpu.VMEM((O, VPAD), jnp.float32),
            pltpu.VMEM((O * VPAD * LANE,), jnp.float32),
            [pltpu.VMEM((CCH, RCH, GRP), jnp.int32) for _ in range(NBUF)],
            [pltpu.VMEM((8 * CCH, RCH, O * 128), jnp.float32)
             for _ in range(NBUF)],
            [pltpu.SemaphoreType.DMA for _ in range(NBUF)],
            [pltpu.SemaphoreType.DMA for _ in range(NBUF)],
        ],
        compiler_params=pltpu.CompilerParams(
            use_tc_tiling_on_sc=False, needs_layout_passes=False),
    )
    def body(t_ref, ids_ref, out_ref, t_v, ts_v, in_v, out_v, in_sem, out_sem):
        wid = lax.axis_index("s") * NC + lax.axis_index("c")
        pltpu.sync_copy(t_ref, t_v)

        def spread(s, carry):
            r0 = s * LANE
            for c in range(O):
                vals16 = t_v[c, pl.ds(r0, LANE)]
                for k in range(LANE):
                    ts_v[pl.ds((r0 + k) * (O * LANE) + c * LANE, LANE)] = (
                        jnp.broadcast_to(vals16[k], (LANE,)))
            return carry

        lax.fori_loop(0, VPAD // LANE, spread, 0)
        offs = [lax.iota(jnp.int32, LANE) + c * LANE for c in range(O)]

        def coords(it):
            rblk0 = wid * RPW + (it // CCH) * RCH
            c0 = (it % CCH) * CCH
            return rblk0, c0

        def start_in(it, buf):
            rblk0, c0 = coords(it)
            pltpu.async_copy(
                ids_ref.at[pl.ds(c0, CCH), pl.ds(rblk0, RCH)],
                in_v[buf], in_sem[buf])

        def wait_in(buf):
            pltpu.make_async_copy(
                ids_ref.at[pl.ds(0, CCH), pl.ds(0, RCH)],
                in_v[buf], in_sem[buf]).wait()

        def start_out(it, buf):
            rblk0, c0 = coords(it)
            pltpu.async_copy(
                out_v[buf],
                out_ref.at[pl.ds(8 * c0, 8 * CCH), pl.ds(rblk0, RCH)],
                out_sem[buf])

        def wait_out(buf):
            pltpu.make_async_copy(
                out_v[buf], out_ref.at[pl.ds(0, 8 * CCH), pl.ds(0, RCH)],
                out_sem[buf]).wait()

        for buf in range(NBUF):
            start_in(buf, buf)

        def step(i, carry):
            for buf in range(NBUF):
                it = NBUF * i + buf
                wait_in(buf)

                @pl.when(it >= NBUF)
                def _():
                    wait_out(buf)

                for ci in range(CCH):
                    for j in range(RCH):
                        @plsc.parallel_loop(0, GRP // LANE, unroll=8)
                        def _(s, _ci=ci, _j=j, _buf=buf):
                            ids16 = in_v[_buf][_ci, _j, pl.ds(s * LANE, LANE)]
                            base = ids16 * (O * LANE)
                            row = _ci * 8 + s // 8
                            b0 = (s % 8) * LANE
                            for c in range(O):
                                vals = plsc.load_gather(
                                    ts_v, [base + offs[c]])
                                out_v[_buf][row, _j,
                                            pl.ds(c * 128 + b0, LANE)] = vals
                start_out(it, buf)

                @pl.when(it + NBUF < ITERS)
                def _():
                    start_in(it + NBUF, buf)

            return carry

        lax.fori_loop(0, ITERS // NBUF, step, 0)
        for buf in range(NBUF):
            wait_out(buf)

    return body(table4, ids_phys)


def kernel(input_ids, emb_table, W, b):
    Bsz, Lseq = input_ids.shape
    V, D = emb_table.shape
    O = W.shape[0]
    emb_pad = jnp.zeros((VPAD, D), jnp.float32).at[:V].set(emb_table)
    table4 = _fold_linear(emb_pad, W, b)
    ids_phys = (input_ids.reshape(RB, 128, CB, 8)
                .transpose(2, 0, 3, 1).reshape(CB, RB, GRP))
    out_phys = _sc_lookup(table4, ids_phys)
    out = (out_phys.reshape(Lseq, RB, O, 128)
           .transpose(1, 3, 0, 2).reshape(Bsz, Lseq, O))
    return out


# speedup vs baseline: 1.1766x; 1.1123x over previous
import functools

import jax
import jax.numpy as jnp
from jax import lax
from jax.experimental import pallas as pl
from jax.experimental.pallas import tpu as pltpu
from jax.experimental.pallas import tpu_sc as plsc

NC, NS = 2, 16
NW = NC * NS
LANE = 16
VPAD = 128

CB = 25
RB = 128
GRP = 1024
CCH = 5
RPW = RB // NW
ITERS = RPW * (CB // CCH)
NBUF = 2


def _fold_linear(emb_pad, W, b):
    O = W.shape[0]

    def body(emb_ref, w_ref, b_ref, t_ref):
        t_ref[...] = lax.dot_general(
            w_ref[...], emb_ref[...], (((1,), (1,)), ((), ())),
            preferred_element_type=jnp.float32) + b_ref[...]

    return pl.pallas_call(
        body,
        out_shape=jax.ShapeDtypeStruct((O, VPAD), jnp.float32),
    )(emb_pad, W, b.reshape(O, 1))


def _sc_lookup(table4, ids_phys):
    O = table4.shape[0]
    mesh = plsc.VectorSubcoreMesh(core_axis_name="c", subcore_axis_name="s")

    @functools.partial(
        pl.kernel,
        out_type=jax.ShapeDtypeStruct((8 * CB, RB, O * 128), jnp.float32),
        mesh=mesh,
        scratch_types=[
            pltpu.VMEM((O, VPAD), jnp.float32),
            pltpu.VMEM((O * VPAD * LANE,), jnp.float32),
            [pltpu.VMEM((CCH, GRP), jnp.int32) for _ in range(NBUF)],
            [pltpu.VMEM((8 * CCH, O * 128), jnp.float32) for _ in range(NBUF)],
            [pltpu.SemaphoreType.DMA for _ in range(NBUF)],
            [pltpu.SemaphoreType.DMA for _ in range(NBUF)],
        ],
        compiler_params=pltpu.CompilerParams(
            use_tc_tiling_on_sc=False, needs_layout_passes=False),
    )
    def body(t_ref, ids_ref, out_ref, t_v, ts_v, in_v, out_v, in_sem, out_sem):
        wid = lax.axis_index("s") * NC + lax.axis_index("c")
        pltpu.sync_copy(t_ref, t_v)

        def spread(s, carry):
            r0 = s * LANE
            for c in range(O):
                vals16 = t_v[c, pl.ds(r0, LANE)]
                for k in range(LANE):
                    ts_v[pl.ds((r0 + k) * (O * LANE) + c * LANE, LANE)] = (
                        jnp.broadcast_to(vals16[k], (LANE,)))
            return carry

        lax.fori_loop(0, VPAD // LANE, spread, 0)
        offs = [lax.iota(jnp.int32, LANE) + c * LANE for c in range(O)]

        def coords(it):
            rblk = wid * RPW + it // CCH
            c0 = (it % CCH) * CCH
            return rblk, c0

        def start_in(it, buf):
            rblk, c0 = coords(it)
            pltpu.async_copy(
                ids_ref.at[pl.ds(c0, CCH), rblk], in_v[buf], in_sem[buf])

        def wait_in(buf):
            pltpu.make_async_copy(
                ids_ref.at[pl.ds(0, CCH), 0], in_v[buf], in_sem[buf]).wait()

        def start_out(it, buf):
            rblk, c0 = coords(it)
            pltpu.async_copy(
                out_v[buf], out_ref.at[pl.ds(8 * c0, 8 * CCH), rblk],
                out_sem[buf])

        def wait_out(buf):
            pltpu.make_async_copy(
                out_v[buf], out_ref.at[pl.ds(0, 8 * CCH), 0],
                out_sem[buf]).wait()

        for buf in range(NBUF):
            start_in(buf, buf)

        def step(i, carry):
            for buf in range(NBUF):
                it = NBUF * i + buf
                wait_in(buf)

                @pl.when(it >= NBUF)
                def _():
                    wait_out(buf)

                for ci in range(CCH):
                    @plsc.parallel_loop(0, GRP // LANE, unroll=8)
                    def _(s, _ci=ci, _buf=buf):
                        ids16 = in_v[_buf][_ci, pl.ds(s * LANE, LANE)]
                        base = ids16 * (O * LANE)
                        row = _ci * 8 + s // 8
                        b0 = (s % 8) * LANE
                        for c in range(O):
                            vals = plsc.load_gather(ts_v, [base + offs[c]])
                            out_v[_buf][row, pl.ds(c * 128 + b0, LANE)] = vals
                start_out(it, buf)

                @pl.when(it + NBUF < ITERS)
                def _():
                    start_in(it + NBUF, buf)

            return carry

        lax.fori_loop(0, ITERS // NBUF, step, 0)
        for buf in range(NBUF):
            wait_out(buf)

    return body(table4, ids_phys)


def kernel(input_ids, emb_table, W, b):
    Bsz, Lseq = input_ids.shape
    V, D = emb_table.shape
    O = W.shape[0]
    emb_pad = jnp.zeros((VPAD, D), jnp.float32).at[:V].set(emb_table)
    table4 = _fold_linear(emb_pad, W, b)
    ids_phys = (input_ids.reshape(RB, 128, CB, 8)
                .transpose(2, 0, 3, 1).reshape(CB, RB, GRP))
    out_phys = _sc_lookup(table4, ids_phys)
    out = (out_phys.reshape(Lseq, RB, O, 128)
           .transpose(1, 3, 0, 2).reshape(Bsz, Lseq, O))
    return out
